# Initial kernel scaffold; baseline (speedup 1.0000x reference)
import jax
import jax.numpy as jnp
from jax.experimental import pallas as pl


def kernel(hidden_states, router_weight, e_bias, gate_w, up_w, down_w, sh_gate, sh_up, sh_down):
    raise NotImplementedError("write your pallas kernel here")


# speedup vs baseline: 1.6578x; 1.6578x over previous
import functools

import jax
import jax.numpy as jnp
from jax import lax
from jax.experimental import pallas as pl
from jax.experimental.pallas import tpu as pltpu
from jax.experimental.pallas import tpu_sc as plsc

B, S, H = 1, 2048, 1024
E, FF = 8, 512
TOPK, NGROUP, TOPKG = 2, 4, 2
SCALE = 2.5
T = B * S
A = T * TOPK
R = 256
NB = A // R + E
NR = NB * R

NC, NS = 2, 16
NW = NC * NS
TPW = T // NW


def _router_kernel(x_ref, w_ref, b_ref, logits_ref, esel_ref, wsel_ref):
    x = x_ref[...]
    w = w_ref[...]
    logits_ref[...] = lax.dot_general(x, w, (((1,), (1,)), ((), ())))
    lt = lax.dot_general(w, x, (((1,), (1,)), ((), ())))
    scores = jax.nn.sigmoid(lt)
    sfc = scores + b_ref[...]

    gs = [sfc[2 * g:2 * g + 1, :] + sfc[2 * g + 1:2 * g + 2, :] for g in range(NGROUP)]
    gmask = []
    for g in range(NGROUP):
        rank = jnp.zeros_like(gs[g])
        for g2 in range(NGROUP):
            beats = (gs[g2] > gs[g]) if g2 > g else (gs[g2] >= gs[g])
            if g2 == g:
                continue
            rank = rank + beats.astype(jnp.float32)
        gmask.append(rank < TOPKG)

    sfcm = [jnp.where(gmask[e // 2], sfc[e:e + 1, :], 0.0) for e in range(E)]
    sel = []
    for e in range(E):
        rank = jnp.zeros_like(sfcm[e])
        for e2 in range(E):
            if e2 == e:
                continue
            beats = (sfcm[e2] > sfcm[e]) if e2 > e else (sfcm[e2] >= sfcm[e])
            rank = rank + beats.astype(jnp.float32)
        sel.append(rank < TOPK)

    sc = [scores[e:e + 1, :] for e in range(E)]
    denom = jnp.zeros_like(sc[0])
    for e in range(E):
        denom = denom + jnp.where(sel[e], sc[e], 0.0)
    wts = [jnp.where(sel[e], sc[e], 0.0) / (denom + 1e-20) * SCALE for e in range(E)]

    e0 = jnp.full_like(sc[0], E, dtype=jnp.int32)
    esum = jnp.zeros_like(e0)
    for e in range(E):
        e0 = jnp.minimum(e0, jnp.where(sel[e], e, E))
        esum = esum + jnp.where(sel[e], e, 0)
    e1 = esum - e0

    w0 = jnp.zeros_like(sc[0])
    w1 = jnp.zeros_like(sc[0])
    for e in range(E):
        w0 = w0 + jnp.where(e0 == e, wts[e], 0.0)
        w1 = w1 + jnp.where(e1 == e, wts[e], 0.0)

    esel_ref[0:1, :] = e0
    esel_ref[1:2, :] = e1
    wsel_ref[0:1, :] = w0
    wsel_ref[1:2, :] = w1


def _router(flat, rw, eb):
    return pl.pallas_call(
        _router_kernel,
        out_shape=(
            jax.ShapeDtypeStruct((T, E), jnp.float32),
            jax.ShapeDtypeStruct((TOPK, T), jnp.int32),
            jax.ShapeDtypeStruct((TOPK, T), jnp.float32),
        ),
    )(flat, rw, eb)


RB = 512
NRB = A // RB


def _rank_kernel(ea_ref, rank_ref, counts_ref, carry_ref):
    b = pl.program_id(0)

    @pl.when(b == 0)
    def _():
        carry_ref[...] = jnp.zeros_like(carry_ref)

    ea = ea_ref[...]
    ioe = lax.broadcasted_iota(jnp.int32, (128, RB), 0)
    onehot = (ioe == ea).astype(jnp.float32)
    ia = lax.broadcasted_iota(jnp.int32, (RB, RB), 0)
    ja = lax.broadcasted_iota(jnp.int32, (RB, RB), 1)
    ustrict = (ia < ja).astype(jnp.float32)
    csum = lax.dot_general(onehot, ustrict, (((1,), (0,)), ((), ())))
    carry = carry_ref[...]
    rank_ref[...] = jnp.sum(onehot * (csum + carry), axis=0, keepdims=True)
    newc = carry + jnp.sum(onehot, axis=1, keepdims=True)
    carry_ref[...] = newc
    counts_ref[...] = newc


def _ranks(easg):
    return pl.pallas_call(
        _rank_kernel,
        grid=(NRB,),
        in_specs=[pl.BlockSpec((1, RB), lambda b: (0, b))],
        out_specs=(
            pl.BlockSpec((1, RB), lambda b: (0, b)),
            pl.BlockSpec((128, 1), lambda b: (0, 0)),
        ),
        out_shape=(
            jax.ShapeDtypeStruct((1, A), jnp.float32),
            jax.ShapeDtypeStruct((128, 1), jnp.float32),
        ),
        scratch_shapes=[pltpu.VMEM((128, 1), jnp.float32)],
    )(easg)


def _pos_kernel(rank_ref, ea_ref, counts_ref, pos_ref, map_ref):
    cnt = counts_ref[...]
    pad = jnp.floor((cnt + (R - 1)) / R) * R
    ie = lax.broadcasted_iota(jnp.int32, (128, 128), 0)
    je = lax.broadcasted_iota(jnp.int32, (128, 128), 1)
    lstrict = (je < ie).astype(jnp.float32)
    base = lax.dot_general(lstrict, pad, (((1,), (0,)), ((), ())))
    ea = ea_ref[...]
    ioe = lax.broadcasted_iota(jnp.int32, (128, A), 0)
    onehot = (ioe == ea).astype(jnp.float32)
    basea = jnp.sum(onehot * base, axis=0, keepdims=True)
    pos_ref[...] = (rank_ref[...] + basea).astype(jnp.int32)
    ends = base + pad
    bstart = lax.broadcasted_iota(jnp.int32, (128, NB), 1).astype(jnp.float32) * R
    mp = jnp.sum((bstart >= ends).astype(jnp.float32), axis=0, keepdims=True)
    map_ref[...] = jnp.minimum(mp.astype(jnp.int32), E - 1)


def _positions(rank, easg, counts):
    return pl.pallas_call(
        _pos_kernel,
        out_shape=(
            jax.ShapeDtypeStruct((1, A), jnp.int32),
            jax.ShapeDtypeStruct((1, NB), jnp.int32),
        ),
    )(rank, easg, counts)


def _dispatch_body(flat_hbm, pos0_hbm, pos1_hbm, xs_hbm, idx0_v, idx1_v, rows_v, sem):
    wid = lax.axis_index("s") * NC + lax.axis_index("c")
    base = wid * TPW
    pltpu.sync_copy(pos0_hbm.at[pl.ds(base, TPW)], idx0_v)
    pltpu.sync_copy(pos1_hbm.at[pl.ds(base, TPW)], idx1_v)
    pltpu.sync_copy(flat_hbm.at[pl.ds(base, TPW)], rows_v)
    pltpu.async_copy(rows_v, xs_hbm.at[idx0_v], sem).wait()
    pltpu.async_copy(rows_v, xs_hbm.at[idx1_v], sem).wait()


def _dispatch(flat, pos0, pos1):
    mesh = plsc.VectorSubcoreMesh(core_axis_name="c", subcore_axis_name="s", num_cores=NC, num_subcores=NS)
    return pl.kernel(
        _dispatch_body,
        out_type=jax.ShapeDtypeStruct((NR, H), jnp.float32),
        mesh=mesh,
        scratch_types=[
            pltpu.VMEM((TPW,), jnp.int32),
            pltpu.VMEM((TPW,), jnp.int32),
            pltpu.VMEM((TPW, H), jnp.float32),
            pltpu.SemaphoreType.DMA,
        ],
    )(flat, pos0, pos1)


def _gmlp_kernel(map_sref, x_ref, g_ref, u_ref, d_ref, y_ref):
    del map_sref
    x = x_ref[...].astype(jnp.bfloat16)
    gw = g_ref[0]
    uw = u_ref[0]
    dw = d_ref[0]
    g = lax.dot_general(x, gw, (((1,), (1,)), ((), ())),
                        preferred_element_type=jnp.float32)
    u = lax.dot_general(x, uw, (((1,), (1,)), ((), ())),
                        preferred_element_type=jnp.float32)
    h = (g * jax.nn.sigmoid(g) * u).astype(jnp.bfloat16)
    y_ref[...] = lax.dot_general(h, dw, (((1,), (1,)), ((), ())),
                                 preferred_element_type=jnp.float32)


def _grouped_mlp(mapv, xs, gw, uw, dw):
    grid_spec = pltpu.PrefetchScalarGridSpec(
        num_scalar_prefetch=1,
        grid=(NB,),
        in_specs=[
            pl.BlockSpec((R, H), lambda b, m: (b, 0)),
            pl.BlockSpec((1, FF, H), lambda b, m: (m[b], 0, 0)),
            pl.BlockSpec((1, FF, H), lambda b, m: (m[b], 0, 0)),
            pl.BlockSpec((1, H, FF), lambda b, m: (m[b], 0, 0)),
        ],
        out_specs=pl.BlockSpec((R, H), lambda b, m: (b, 0)),
    )
    return pl.pallas_call(
        _gmlp_kernel,
        grid_spec=grid_spec,
        out_shape=jax.ShapeDtypeStruct((NR, H), jnp.float32),
    )(mapv, xs, gw, uw, dw)


def _combine_body(ys_hbm, pos0_hbm, pos1_hbm, y0_hbm, y1_hbm, idx_v, buf_v, sem):
    wid = lax.axis_index("s") * NC + lax.axis_index("c")
    base = wid * TPW
    pltpu.sync_copy(pos0_hbm.at[pl.ds(base, TPW)], idx_v)
    pltpu.async_copy(ys_hbm.at[idx_v], buf_v, sem).wait()
    pltpu.sync_copy(buf_v, y0_hbm.at[pl.ds(base, TPW)])
    pltpu.sync_copy(pos1_hbm.at[pl.ds(base, TPW)], idx_v)
    pltpu.async_copy(ys_hbm.at[idx_v], buf_v, sem).wait()
    pltpu.sync_copy(buf_v, y1_hbm.at[pl.ds(base, TPW)])


def _combine(ys, pos0, pos1):
    mesh = plsc.VectorSubcoreMesh(core_axis_name="c", subcore_axis_name="s", num_cores=NC, num_subcores=NS)
    return pl.kernel(
        _combine_body,
        out_type=(
            jax.ShapeDtypeStruct((T, H), jnp.float32),
            jax.ShapeDtypeStruct((T, H), jnp.float32),
        ),
        mesh=mesh,
        scratch_types=[
            pltpu.VMEM((TPW,), jnp.int32),
            pltpu.VMEM((TPW, H), jnp.float32),
            pltpu.SemaphoreType.DMA,
        ],
    )(ys, pos0, pos1)


TB = 256


def _final_kernel(x_ref, g_ref, u_ref, d_ref, y0_ref, y1_ref, w_ref, o_ref):
    x = x_ref[...].astype(jnp.bfloat16)
    g = lax.dot_general(x, g_ref[...], (((1,), (1,)), ((), ())),
                        preferred_element_type=jnp.float32)
    u = lax.dot_general(x, u_ref[...], (((1,), (1,)), ((), ())),
                        preferred_element_type=jnp.float32)
    h = (g * jax.nn.sigmoid(g) * u).astype(jnp.bfloat16)
    sh = lax.dot_general(h, d_ref[...], (((1,), (1,)), ((), ())),
                         preferred_element_type=jnp.float32)
    ii = lax.broadcasted_iota(jnp.int32, (TB, TB), 0)
    jj = lax.broadcasted_iota(jnp.int32, (TB, TB), 1)
    eye = (ii == jj).astype(jnp.float32)
    wt = lax.dot_general(eye, w_ref[...], (((1,), (1,)), ((), ())),
                         precision=lax.Precision.HIGHEST)
    o_ref[...] = sh + wt[:, 0:1] * y0_ref[...] + wt[:, 1:2] * y1_ref[...]


def _final(flat, shg, shu, shd, y0g, y1g, wsel):
    return pl.pallas_call(
        _final_kernel,
        grid=(T // TB,),
        in_specs=[
            pl.BlockSpec((TB, H), lambda b: (b, 0)),
            pl.BlockSpec((FF, H), lambda b: (0, 0)),
            pl.BlockSpec((FF, H), lambda b: (0, 0)),
            pl.BlockSpec((H, FF), lambda b: (0, 0)),
            pl.BlockSpec((TB, H), lambda b: (b, 0)),
            pl.BlockSpec((TB, H), lambda b: (b, 0)),
            pl.BlockSpec((TOPK, TB), lambda b: (0, b)),
        ],
        out_specs=pl.BlockSpec((TB, H), lambda b: (b, 0)),
        out_shape=jax.ShapeDtypeStruct((T, H), jnp.float32),
    )(flat, shg, shu, shd, y0g, y1g, wsel)


def kernel(hidden_states, router_weight, e_bias, gate_w, up_w, down_w,
           sh_gate, sh_up, sh_down):
    flat = hidden_states.reshape(T, H)
    logits, esel, wsel = _router(flat, router_weight, e_bias.reshape(E, 1))
    easg = esel.reshape(1, A)
    rank, counts = _ranks(easg)
    pos, mapv = _positions(rank, easg, counts)
    posf = pos.reshape(A)
    pos0, pos1 = posf[:T], posf[T:]
    xs = _dispatch(flat, pos0, pos1)
    ys = _grouped_mlp(mapv.reshape(NB), xs,
                      gate_w.astype(jnp.bfloat16), up_w.astype(jnp.bfloat16),
                      down_w.astype(jnp.bfloat16))
    y0g, y1g = _combine(ys, pos0, pos1)
    out = _final(flat, sh_gate.astype(jnp.bfloat16), sh_up.astype(jnp.bfloat16),
                 sh_down.astype(jnp.bfloat16), y0g, y1g, wsel)
    return out.reshape(B, S, H), logits


# speedup vs baseline: 1.8368x; 1.1080x over previous
import functools

import jax
import jax.numpy as jnp
from jax import lax
from jax.experimental import pallas as pl
from jax.experimental.pallas import tpu as pltpu
from jax.experimental.pallas import tpu_sc as plsc

B, S, H = 1, 2048, 1024
E, FF = 8, 512
TOPK, NGROUP, TOPKG = 2, 4, 2
SCALE = 2.5
T = B * S
A = T * TOPK
R = 256
NB = A // R + E
NR = NB * R

NC, NS = 2, 16
NW = NC * NS
TPW = T // NW


def _router_kernel(x_ref, w_ref, b_ref, logits_ref, esel_ref, wsel_ref):
    x = x_ref[...]
    w = w_ref[...]
    logits_ref[...] = lax.dot_general(x, w, (((1,), (1,)), ((), ())))
    lt = lax.dot_general(w, x, (((1,), (1,)), ((), ())))
    scores = jax.nn.sigmoid(lt)
    sfc = scores + b_ref[...]

    gs = [sfc[2 * g:2 * g + 1, :] + sfc[2 * g + 1:2 * g + 2, :] for g in range(NGROUP)]
    gmask = []
    for g in range(NGROUP):
        rank = jnp.zeros_like(gs[g])
        for g2 in range(NGROUP):
            beats = (gs[g2] > gs[g]) if g2 > g else (gs[g2] >= gs[g])
            if g2 == g:
                continue
            rank = rank + beats.astype(jnp.float32)
        gmask.append(rank < TOPKG)

    sfcm = [jnp.where(gmask[e // 2], sfc[e:e + 1, :], 0.0) for e in range(E)]
    sel = []
    for e in range(E):
        rank = jnp.zeros_like(sfcm[e])
        for e2 in range(E):
            if e2 == e:
                continue
            beats = (sfcm[e2] > sfcm[e]) if e2 > e else (sfcm[e2] >= sfcm[e])
            rank = rank + beats.astype(jnp.float32)
        sel.append(rank < TOPK)

    sc = [scores[e:e + 1, :] for e in range(E)]
    denom = jnp.zeros_like(sc[0])
    for e in range(E):
        denom = denom + jnp.where(sel[e], sc[e], 0.0)
    wts = [jnp.where(sel[e], sc[e], 0.0) / (denom + 1e-20) * SCALE for e in range(E)]

    e0 = jnp.full_like(sc[0], E, dtype=jnp.int32)
    esum = jnp.zeros_like(e0)
    for e in range(E):
        e0 = jnp.minimum(e0, jnp.where(sel[e], e, E))
        esum = esum + jnp.where(sel[e], e, 0)
    e1 = esum - e0

    w0 = jnp.zeros_like(sc[0])
    w1 = jnp.zeros_like(sc[0])
    for e in range(E):
        w0 = w0 + jnp.where(e0 == e, wts[e], 0.0)
        w1 = w1 + jnp.where(e1 == e, wts[e], 0.0)

    esel_ref[0:1, :] = e0
    esel_ref[1:2, :] = e1
    wsel_ref[0:1, :] = w0
    wsel_ref[1:2, :] = w1


def _router(flat, rw, eb):
    return pl.pallas_call(
        _router_kernel,
        out_shape=(
            jax.ShapeDtypeStruct((T, E), jnp.float32),
            jax.ShapeDtypeStruct((TOPK, T), jnp.int32),
            jax.ShapeDtypeStruct((TOPK, T), jnp.float32),
        ),
    )(flat, rw, eb)


RB = 512
NRB = A // RB


def _rank_kernel(ea_ref, rank_ref, counts_ref, carry_ref):
    b = pl.program_id(0)

    @pl.when(b == 0)
    def _():
        carry_ref[...] = jnp.zeros_like(carry_ref)

    ea = ea_ref[...]
    ioe = lax.broadcasted_iota(jnp.int32, (128, RB), 0)
    onehot = (ioe == ea).astype(jnp.float32)
    ia = lax.broadcasted_iota(jnp.int32, (RB, RB), 0)
    ja = lax.broadcasted_iota(jnp.int32, (RB, RB), 1)
    ustrict = (ia < ja).astype(jnp.float32)
    csum = lax.dot_general(onehot, ustrict, (((1,), (0,)), ((), ())))
    carry = carry_ref[...]
    rank_ref[...] = jnp.sum(onehot * (csum + carry), axis=0, keepdims=True)
    newc = carry + jnp.sum(onehot, axis=1, keepdims=True)
    carry_ref[...] = newc
    counts_ref[...] = newc


def _ranks(easg):
    return pl.pallas_call(
        _rank_kernel,
        grid=(NRB,),
        in_specs=[pl.BlockSpec((1, RB), lambda b: (0, b))],
        out_specs=(
            pl.BlockSpec((1, RB), lambda b: (0, b)),
            pl.BlockSpec((128, 1), lambda b: (0, 0)),
        ),
        out_shape=(
            jax.ShapeDtypeStruct((1, A), jnp.float32),
            jax.ShapeDtypeStruct((128, 1), jnp.float32),
        ),
        scratch_shapes=[pltpu.VMEM((128, 1), jnp.float32)],
    )(easg)


def _pos_kernel(rank_ref, ea_ref, counts_ref, pos_ref, map_ref):
    cnt = counts_ref[...]
    pad = jnp.floor((cnt + (R - 1)) / R) * R
    ie = lax.broadcasted_iota(jnp.int32, (128, 128), 0)
    je = lax.broadcasted_iota(jnp.int32, (128, 128), 1)
    lstrict = (je < ie).astype(jnp.float32)
    base = lax.dot_general(lstrict, pad, (((1,), (0,)), ((), ())))
    ea = ea_ref[...]
    ioe = lax.broadcasted_iota(jnp.int32, (128, A), 0)
    onehot = (ioe == ea).astype(jnp.float32)
    basea = jnp.sum(onehot * base, axis=0, keepdims=True)
    pos_ref[...] = (rank_ref[...] + basea).astype(jnp.int32)
    ends = base + pad
    bstart = lax.broadcasted_iota(jnp.int32, (128, NB), 1).astype(jnp.float32) * R
    mp = jnp.sum((bstart >= ends).astype(jnp.float32), axis=0, keepdims=True)
    map_ref[...] = jnp.minimum(mp.astype(jnp.int32), E - 1)


def _positions(rank, easg, counts):
    return pl.pallas_call(
        _pos_kernel,
        out_shape=(
            jax.ShapeDtypeStruct((1, A), jnp.int32),
            jax.ShapeDtypeStruct((1, NB), jnp.int32),
        ),
    )(rank, easg, counts)


def _dispatch_body(flat_hbm, pos0_hbm, pos1_hbm, xs_hbm,
                   idx0_v, idx1_v, rows_v, sem_i0, sem_i1, sem_r, sem_s0, sem_s1):
    wid = lax.axis_index("s") * NC + lax.axis_index("c")
    base = wid * TPW
    c_i0 = pltpu.async_copy(pos0_hbm.at[pl.ds(base, TPW)], idx0_v, sem_i0)
    c_i1 = pltpu.async_copy(pos1_hbm.at[pl.ds(base, TPW)], idx1_v, sem_i1)
    c_r = pltpu.async_copy(flat_hbm.at[pl.ds(base, TPW)], rows_v, sem_r)
    c_i0.wait()
    c_r.wait()
    s0 = pltpu.async_copy(rows_v, xs_hbm.at[idx0_v], sem_s0)
    c_i1.wait()
    s1 = pltpu.async_copy(rows_v, xs_hbm.at[idx1_v], sem_s1)
    s0.wait()
    s1.wait()


def _dispatch(flat, pos0, pos1):
    mesh = plsc.VectorSubcoreMesh(core_axis_name="c", subcore_axis_name="s", num_cores=NC, num_subcores=NS)
    return pl.kernel(
        _dispatch_body,
        out_type=jax.ShapeDtypeStruct((NR, H), jnp.float32),
        mesh=mesh,
        scratch_types=[
            pltpu.VMEM((TPW,), jnp.int32),
            pltpu.VMEM((TPW,), jnp.int32),
            pltpu.VMEM((TPW, H), jnp.float32),
            pltpu.SemaphoreType.DMA,
            pltpu.SemaphoreType.DMA,
            pltpu.SemaphoreType.DMA,
            pltpu.SemaphoreType.DMA,
            pltpu.SemaphoreType.DMA,
        ],
    )(flat, pos0, pos1)


def _gmlp_kernel(map_sref, x_ref, g_ref, u_ref, d_ref, y_ref):
    del map_sref
    x = x_ref[...]
    gw = g_ref[0]
    uw = u_ref[0]
    dw = d_ref[0]
    g = lax.dot_general(x, gw, (((1,), (1,)), ((), ())))
    u = lax.dot_general(x, uw, (((1,), (1,)), ((), ())))
    h = g * jax.nn.sigmoid(g) * u
    y_ref[...] = lax.dot_general(h, dw, (((1,), (1,)), ((), ())))


def _grouped_mlp(mapv, xs, gw, uw, dw):
    grid_spec = pltpu.PrefetchScalarGridSpec(
        num_scalar_prefetch=1,
        grid=(NB,),
        in_specs=[
            pl.BlockSpec((R, H), lambda b, m: (b, 0)),
            pl.BlockSpec((1, FF, H), lambda b, m: (m[b], 0, 0)),
            pl.BlockSpec((1, FF, H), lambda b, m: (m[b], 0, 0)),
            pl.BlockSpec((1, H, FF), lambda b, m: (m[b], 0, 0)),
        ],
        out_specs=pl.BlockSpec((R, H), lambda b, m: (b, 0)),
    )
    return pl.pallas_call(
        _gmlp_kernel,
        grid_spec=grid_spec,
        out_shape=jax.ShapeDtypeStruct((NR, H), jnp.float32),
    )(mapv, xs, gw, uw, dw)


HPW = TPW // 2


def _combine_body(ys_hbm, pos0_hbm, pos1_hbm, y0_hbm, y1_hbm,
                  idx0_v, idx1_v, b0, b1, b2,
                  gs0, gs1, gs2, os0, os1, os2, sem_i):
    wid = lax.axis_index("s") * NC + lax.axis_index("c")
    base = wid * TPW
    ci0 = pltpu.async_copy(pos0_hbm.at[pl.ds(base, TPW)], idx0_v, sem_i)
    ci1 = pltpu.async_copy(pos1_hbm.at[pl.ds(base, TPW)], idx1_v, sem_i)
    ci0.wait()
    ci1.wait()
    g0 = pltpu.async_copy(ys_hbm.at[idx0_v.at[pl.ds(0, HPW)]], b0, gs0)
    g1 = pltpu.async_copy(ys_hbm.at[idx0_v.at[pl.ds(HPW, HPW)]], b1, gs1)
    g2 = pltpu.async_copy(ys_hbm.at[idx1_v.at[pl.ds(0, HPW)]], b2, gs2)
    g0.wait()
    o0 = pltpu.async_copy(b0, y0_hbm.at[pl.ds(base, HPW)], os0)
    g1.wait()
    o1 = pltpu.async_copy(b1, y0_hbm.at[pl.ds(base + HPW, HPW)], os1)
    o0.wait()
    g3 = pltpu.async_copy(ys_hbm.at[idx1_v.at[pl.ds(HPW, HPW)]], b0, gs0)
    g2.wait()
    o2 = pltpu.async_copy(b2, y1_hbm.at[pl.ds(base, HPW)], os2)
    g3.wait()
    o3 = pltpu.async_copy(b0, y1_hbm.at[pl.ds(base + HPW, HPW)], os0)
    o1.wait()
    o2.wait()
    o3.wait()


def _combine(ys, pos0, pos1):
    mesh = plsc.VectorSubcoreMesh(core_axis_name="c", subcore_axis_name="s", num_cores=NC, num_subcores=NS)
    return pl.kernel(
        _combine_body,
        out_type=(
            jax.ShapeDtypeStruct((T, H), jnp.float32),
            jax.ShapeDtypeStruct((T, H), jnp.float32),
        ),
        mesh=mesh,
        scratch_types=[
            pltpu.VMEM((TPW,), jnp.int32),
            pltpu.VMEM((TPW,), jnp.int32),
            pltpu.VMEM((HPW, H), jnp.float32),
            pltpu.VMEM((HPW, H), jnp.float32),
            pltpu.VMEM((HPW, H), jnp.float32),
            pltpu.SemaphoreType.DMA,
            pltpu.SemaphoreType.DMA,
            pltpu.SemaphoreType.DMA,
            pltpu.SemaphoreType.DMA,
            pltpu.SemaphoreType.DMA,
            pltpu.SemaphoreType.DMA,
            pltpu.SemaphoreType.DMA,
        ],
    )(ys, pos0, pos1)


TB = 256


def _final_kernel(x_ref, g_ref, u_ref, d_ref, y0_ref, y1_ref, w_ref, o_ref):
    x = x_ref[...]
    g = lax.dot_general(x, g_ref[...], (((1,), (1,)), ((), ())))
    u = lax.dot_general(x, u_ref[...], (((1,), (1,)), ((), ())))
    h = g * jax.nn.sigmoid(g) * u
    sh = lax.dot_general(h, d_ref[...], (((1,), (1,)), ((), ())))
    ii = lax.broadcasted_iota(jnp.int32, (TB, TB), 0)
    jj = lax.broadcasted_iota(jnp.int32, (TB, TB), 1)
    eye = (ii == jj).astype(jnp.float32)
    wt = lax.dot_general(eye, w_ref[...], (((1,), (1,)), ((), ())),
                         precision=lax.Precision.HIGHEST)
    o_ref[...] = sh + wt[:, 0:1] * y0_ref[...] + wt[:, 1:2] * y1_ref[...]


def _final(flat, shg, shu, shd, y0g, y1g, wsel):
    return pl.pallas_call(
        _final_kernel,
        grid=(T // TB,),
        in_specs=[
            pl.BlockSpec((TB, H), lambda b: (b, 0)),
            pl.BlockSpec((FF, H), lambda b: (0, 0)),
            pl.BlockSpec((FF, H), lambda b: (0, 0)),
            pl.BlockSpec((H, FF), lambda b: (0, 0)),
            pl.BlockSpec((TB, H), lambda b: (b, 0)),
            pl.BlockSpec((TB, H), lambda b: (b, 0)),
            pl.BlockSpec((TOPK, TB), lambda b: (0, b)),
        ],
        out_specs=pl.BlockSpec((TB, H), lambda b: (b, 0)),
        out_shape=jax.ShapeDtypeStruct((T, H), jnp.float32),
    )(flat, shg, shu, shd, y0g, y1g, wsel)


def kernel(hidden_states, router_weight, e_bias, gate_w, up_w, down_w,
           sh_gate, sh_up, sh_down):
    flat = hidden_states.reshape(T, H)
    logits, esel, wsel = _router(flat, router_weight, e_bias.reshape(E, 1))
    easg = esel.reshape(1, A)
    rank, counts = _ranks(easg)
    pos, mapv = _positions(rank, easg, counts)
    posf = pos.reshape(A)
    pos0, pos1 = posf[:T], posf[T:]
    xs = _dispatch(flat, pos0, pos1)
    ys = _grouped_mlp(mapv.reshape(NB), xs, gate_w, up_w, down_w)
    y0g, y1g = _combine(ys, pos0, pos1)
    out = _final(flat, sh_gate, sh_up, sh_down, y0g, y1g, wsel)
    return out.reshape(B, S, H), logits
